# baseline (speedup 1.0000x reference)
import jax
import jax.numpy as jnp
from jax.experimental import pallas as pl


_BN = 512


def _normalize(qb):
    nrm = jnp.sqrt(jnp.sum(qb * qb, axis=1, keepdims=True))
    return qb / jnp.maximum(nrm, 1e-12)


def _pass1_kernel(q_ref, mem_ref, smmem_ref, upd_ref, cmax_ref, csum_ref):
    i = pl.program_id(0)
    mem = mem_ref[:]
    qn = _normalize(q_ref[:])
    s = jax.lax.dot_general(
        qn, mem, (((1,), (1,)), ((), ())),
        precision=jax.lax.Precision.HIGHEST,
        preferred_element_type=jnp.float32,
    )
    rmax = jnp.max(s, axis=1, keepdims=True)
    e = jnp.exp(s - rmax)
    p = e / jnp.sum(e, axis=1, keepdims=True)
    smmem_ref[:] = p
    upd_ref[:] = jax.lax.dot_general(
        p, mem, (((1,), (0,)), ((), ())),
        precision=jax.lax.Precision.HIGHEST,
        preferred_element_type=jnp.float32,
    )
    @pl.when(i == 0)
    def _init():
        cmax_ref[:] = jnp.full_like(cmax_ref, -jnp.inf)
        csum_ref[:] = jnp.zeros_like(csum_ref)

    m_old = cmax_ref[:]
    bmax = jnp.max(s, axis=0, keepdims=True)
    m_new = jnp.maximum(m_old, bmax)
    bsum = jnp.sum(jnp.exp(s - m_new), axis=0, keepdims=True)
    csum_ref[:] = csum_ref[:] * jnp.exp(m_old - m_new) + bsum
    cmax_ref[:] = m_new


def _pass2_kernel(q_ref, mem_ref, cmax_ref, csum_ref, smq_ref):
    qn = _normalize(q_ref[:])
    s = jax.lax.dot_general(
        qn, mem_ref[:], (((1,), (1,)), ((), ())),
        precision=jax.lax.Precision.HIGHEST,
        preferred_element_type=jnp.float32,
    )
    smq_ref[:] = jnp.exp(s - cmax_ref[:]) / csum_ref[:]


@jax.jit
def _memory_read(query_source, fusion_keys):
    b, d, h, w = query_source.shape
    m = fusion_keys.shape[0]
    n = b * h * w
    nb = n // _BN
    q = jnp.transpose(query_source, (0, 2, 3, 1)).reshape(n, d)

    smmem, upd, cmax, csum = pl.pallas_call(
        _pass1_kernel,
        grid=(nb,),
        in_specs=[
            pl.BlockSpec((_BN, d), lambda i: (i, 0)),
            pl.BlockSpec((m, d), lambda i: (0, 0)),
        ],
        out_specs=[
            pl.BlockSpec((_BN, m), lambda i: (i, 0)),
            pl.BlockSpec((_BN, d), lambda i: (i, 0)),
            pl.BlockSpec((1, m), lambda i: (0, 0)),
            pl.BlockSpec((1, m), lambda i: (0, 0)),
        ],
        out_shape=[
            jax.ShapeDtypeStruct((n, m), jnp.float32),
            jax.ShapeDtypeStruct((n, d), jnp.float32),
            jax.ShapeDtypeStruct((1, m), jnp.float32),
            jax.ShapeDtypeStruct((1, m), jnp.float32),
        ],
    )(q, fusion_keys)

    smq = pl.pallas_call(
        _pass2_kernel,
        grid=(nb,),
        in_specs=[
            pl.BlockSpec((_BN, d), lambda i: (i, 0)),
            pl.BlockSpec((m, d), lambda i: (0, 0)),
            pl.BlockSpec((1, m), lambda i: (0, 0)),
            pl.BlockSpec((1, m), lambda i: (0, 0)),
        ],
        out_specs=pl.BlockSpec((_BN, m), lambda i: (i, 0)),
        out_shape=jax.ShapeDtypeStruct((n, m), jnp.float32),
    )(q, fusion_keys, cmax, csum)

    updated_query = jnp.transpose(upd.reshape(b, h, w, d), (0, 3, 1, 2))
    return updated_query, smq, smmem


def kernel(query_source, keys, only_update, fusion_keys):
    return _memory_read(query_source, fusion_keys)


# speedup vs baseline: 3.2115x; 3.2115x over previous
import jax
import jax.numpy as jnp
from jax.experimental import pallas as pl


_BN = 512


def _normalize(qb):
    nrm = jnp.sqrt(jnp.sum(qb * qb, axis=1, keepdims=True))
    return qb / jnp.maximum(nrm, 1e-12)


def _pass1_kernel(q_ref, mem_ref, smmem_ref, upd_ref, csum_ref):
    i = pl.program_id(0)
    mem = mem_ref[:]
    qn = _normalize(q_ref[:])
    s = jax.lax.dot_general(
        qn, mem, (((1,), (1,)), ((), ())),
        preferred_element_type=jnp.float32,
    )
    e = jnp.exp(s)
    rsum = jnp.sum(e, axis=1, keepdims=True)
    p = e * (1.0 / rsum)
    smmem_ref[:] = p
    upd_ref[:] = jax.lax.dot_general(
        p, mem, (((1,), (0,)), ((), ())),
        preferred_element_type=jnp.float32,
    )

    @pl.when(i == 0)
    def _init():
        csum_ref[:] = jnp.zeros_like(csum_ref)

    csum_ref[:] += jnp.sum(e, axis=0, keepdims=True)


def _pass2_kernel(q_ref, mem_ref, csum_ref, smq_ref):
    qn = _normalize(q_ref[:])
    s = jax.lax.dot_general(
        qn, mem_ref[:], (((1,), (1,)), ((), ())),
        preferred_element_type=jnp.float32,
    )
    smq_ref[:] = jnp.exp(s) * (1.0 / csum_ref[:])


@jax.jit
def _memory_read(query_source, fusion_keys):
    b, d, h, w = query_source.shape
    m = fusion_keys.shape[0]
    n = b * h * w
    nb = n // _BN
    q = jnp.transpose(query_source, (0, 2, 3, 1)).reshape(n, d)

    smmem, upd, csum = pl.pallas_call(
        _pass1_kernel,
        grid=(nb,),
        in_specs=[
            pl.BlockSpec((_BN, d), lambda i: (i, 0)),
            pl.BlockSpec((m, d), lambda i: (0, 0)),
        ],
        out_specs=[
            pl.BlockSpec((_BN, m), lambda i: (i, 0)),
            pl.BlockSpec((_BN, d), lambda i: (i, 0)),
            pl.BlockSpec((1, m), lambda i: (0, 0)),
        ],
        out_shape=[
            jax.ShapeDtypeStruct((n, m), jnp.float32),
            jax.ShapeDtypeStruct((n, d), jnp.float32),
            jax.ShapeDtypeStruct((1, m), jnp.float32),
        ],
    )(q, fusion_keys)

    smq = pl.pallas_call(
        _pass2_kernel,
        grid=(nb,),
        in_specs=[
            pl.BlockSpec((_BN, d), lambda i: (i, 0)),
            pl.BlockSpec((m, d), lambda i: (0, 0)),
            pl.BlockSpec((1, m), lambda i: (0, 0)),
        ],
        out_specs=pl.BlockSpec((_BN, m), lambda i: (i, 0)),
        out_shape=jax.ShapeDtypeStruct((n, m), jnp.float32),
    )(q, fusion_keys, csum)

    updated_query = jnp.transpose(upd.reshape(b, h, w, d), (0, 3, 1, 2))
    return updated_query, smq, smmem


def kernel(query_source, keys, only_update, fusion_keys):
    return _memory_read(query_source, fusion_keys)


# speedup vs baseline: 3.3290x; 1.0366x over previous
import jax
import jax.numpy as jnp
from jax.experimental import pallas as pl


_BW = 512


def _normalize_cols(qb):
    nrm = jnp.sqrt(jnp.sum(qb * qb, axis=0, keepdims=True))
    return qb * (1.0 / jnp.maximum(nrm, 1e-12))


def _pass1_kernel(q_ref, mem_ref, smmem_ref, upd_ref, csum_ref):
    b = pl.program_id(0)
    j = pl.program_id(1)
    mem = mem_ref[:]
    qn = _normalize_cols(q_ref[0])
    s = jax.lax.dot_general(
        qn, mem, (((0,), (1,)), ((), ())),
        preferred_element_type=jnp.float32,
    )
    e = jnp.exp(s)
    rsum = jnp.sum(e, axis=1, keepdims=True)
    p = e * (1.0 / rsum)
    smmem_ref[:] = p
    upd_ref[0] = jax.lax.dot_general(
        mem, p, (((0,), (1,)), ((), ())),
        preferred_element_type=jnp.float32,
    )

    @pl.when(jnp.logical_and(b == 0, j == 0))
    def _init():
        csum_ref[:] = jnp.zeros_like(csum_ref)

    csum_ref[:] += jnp.sum(e, axis=0, keepdims=True)


def _pass2_kernel(q_ref, mem_ref, csum_ref, smq_ref):
    qn = _normalize_cols(q_ref[0])
    s = jax.lax.dot_general(
        qn, mem_ref[:], (((0,), (1,)), ((), ())),
        preferred_element_type=jnp.float32,
    )
    smq_ref[:] = jnp.exp(s) * (1.0 / csum_ref[:])


@jax.jit
def _memory_read(query_source, fusion_keys):
    b, d, h, w = query_source.shape
    m = fusion_keys.shape[0]
    hw = h * w
    n = b * hw
    jb = hw // _BW
    q = query_source.reshape(b, d, hw)

    smmem, upd, csum = pl.pallas_call(
        _pass1_kernel,
        grid=(b, jb),
        in_specs=[
            pl.BlockSpec((1, d, _BW), lambda bi, ji: (bi, 0, ji)),
            pl.BlockSpec((m, d), lambda bi, ji: (0, 0)),
        ],
        out_specs=[
            pl.BlockSpec((_BW, m), lambda bi, ji: (bi * jb + ji, 0)),
            pl.BlockSpec((1, d, _BW), lambda bi, ji: (bi, 0, ji)),
            pl.BlockSpec((1, m), lambda bi, ji: (0, 0)),
        ],
        out_shape=[
            jax.ShapeDtypeStruct((n, m), jnp.float32),
            jax.ShapeDtypeStruct((b, d, hw), jnp.float32),
            jax.ShapeDtypeStruct((1, m), jnp.float32),
        ],
    )(q, fusion_keys)

    smq = pl.pallas_call(
        _pass2_kernel,
        grid=(b, jb),
        in_specs=[
            pl.BlockSpec((1, d, _BW), lambda bi, ji: (bi, 0, ji)),
            pl.BlockSpec((m, d), lambda bi, ji: (0, 0)),
            pl.BlockSpec((1, m), lambda bi, ji: (0, 0)),
        ],
        out_specs=pl.BlockSpec((_BW, m), lambda bi, ji: (bi * jb + ji, 0)),
        out_shape=jax.ShapeDtypeStruct((n, m), jnp.float32),
    )(q, fusion_keys, csum)

    updated_query = upd.reshape(b, d, h, w)
    return updated_query, smq, smmem


def kernel(query_source, keys, only_update, fusion_keys):
    return _memory_read(query_source, fusion_keys)


# speedup vs baseline: 4.4783x; 1.3452x over previous
import jax
import jax.numpy as jnp
from jax.experimental import pallas as pl
from jax.experimental.pallas import tpu as pltpu


_BW = 2304


def _normalize_cols(qb):
    nrm = jnp.sqrt(jnp.sum(qb * qb, axis=0, keepdims=True))
    return qb * (1.0 / jnp.maximum(nrm, 1e-12))


def _pass1_kernel(q_ref, mem_ref, smmem_ref, upd_ref, csum_ref):
    j = pl.program_id(1)
    mem = mem_ref[:]
    qn = _normalize_cols(q_ref[0])
    s = jax.lax.dot_general(
        qn, mem, (((0,), (1,)), ((), ())),
        preferred_element_type=jnp.float32,
    )
    e = jnp.exp(s)
    rsum = jnp.sum(e, axis=1, keepdims=True)
    rrec = 1.0 / rsum
    smmem_ref[:] = e * rrec
    eT = jax.lax.dot_general(
        mem, e, (((0,), (1,)), ((), ())),
        preferred_element_type=jnp.float32,
    )
    upd_ref[0] = eT * rrec.reshape(1, -1)

    @pl.when(j == 0)
    def _init():
        csum_ref[:] = jnp.zeros_like(csum_ref)

    csum_ref[0] += jnp.sum(e, axis=0, keepdims=True)


def _pass2_kernel(q_ref, mem_ref, csum_ref, smq_ref):
    qn = _normalize_cols(q_ref[0])
    s = jax.lax.dot_general(
        qn, mem_ref[:], (((0,), (1,)), ((), ())),
        preferred_element_type=jnp.float32,
    )
    csum = jnp.sum(csum_ref[:, 0, :], axis=0, keepdims=True)
    smq_ref[:] = jnp.exp(s) * (1.0 / csum)


@jax.jit
def _memory_read(query_source, fusion_keys):
    b, d, h, w = query_source.shape
    m = fusion_keys.shape[0]
    hw = h * w
    n = b * hw
    jb = hw // _BW
    q = query_source.reshape(b, d, hw)

    smmem, upd, csum = pl.pallas_call(
        _pass1_kernel,
        grid=(b, jb),
        in_specs=[
            pl.BlockSpec((1, d, _BW), lambda bi, ji: (bi, 0, ji)),
            pl.BlockSpec((m, d), lambda bi, ji: (0, 0)),
        ],
        out_specs=[
            pl.BlockSpec((_BW, m), lambda bi, ji: (bi * jb + ji, 0)),
            pl.BlockSpec((1, d, _BW), lambda bi, ji: (bi, 0, ji)),
            pl.BlockSpec((1, 1, m), lambda bi, ji: (bi, 0, 0)),
        ],
        out_shape=[
            jax.ShapeDtypeStruct((n, m), jnp.float32),
            jax.ShapeDtypeStruct((b, d, hw), jnp.float32),
            jax.ShapeDtypeStruct((b, 1, m), jnp.float32),
        ],
        compiler_params=pltpu.CompilerParams(
            dimension_semantics=("parallel", "arbitrary"),
        ),
    )(q, fusion_keys)

    smq = pl.pallas_call(
        _pass2_kernel,
        grid=(b, jb),
        in_specs=[
            pl.BlockSpec((1, d, _BW), lambda bi, ji: (bi, 0, ji)),
            pl.BlockSpec((m, d), lambda bi, ji: (0, 0)),
            pl.BlockSpec((b, 1, m), lambda bi, ji: (0, 0, 0)),
        ],
        out_specs=pl.BlockSpec((_BW, m), lambda bi, ji: (bi * jb + ji, 0)),
        out_shape=jax.ShapeDtypeStruct((n, m), jnp.float32),
        compiler_params=pltpu.CompilerParams(
            dimension_semantics=("parallel", "parallel"),
        ),
    )(q, fusion_keys, csum)

    updated_query = upd.reshape(b, d, h, w)
    return updated_query, smq, smmem


def kernel(query_source, keys, only_update, fusion_keys):
    return _memory_read(query_source, fusion_keys)
